# baseline (device time: 208707 ns/iter reference)
import jax
import jax.numpy as jnp
from jax import lax
from jax.experimental import pallas as pl
from jax.experimental.pallas import tpu as pltpu

T = 2048
D = 4096
V_LOC = 8192
Q = V_LOC // 4
V_TILE = 128
N_TILES = Q // V_TILE


def _stats_body(q_ref, x_ref, w_ref, lab_ref, acc_ref, xb_ref):
    j = pl.program_id(0)
    my_x = lax.axis_index("x")

    @pl.when(j == 0)
    def _():
        def _conv(i, carry):
            rs = pl.ds(i * 256, 256)
            xb_ref[rs, :] = x_ref[rs, :].astype(jnp.bfloat16)
            return carry

        lax.fori_loop(0, T // 256, _conv, 0)
        acc_ref[...] = jnp.zeros((T, 8), jnp.float32)
        acc_ref[:, 0:1] = jnp.full((T, 1), -jnp.inf, jnp.float32)

    logits = jnp.dot(
        xb_ref[...],
        w_ref[...].astype(jnp.bfloat16),
        preferred_element_type=jnp.float32,
    )

    tile_m = jnp.max(logits, axis=1, keepdims=True)
    m_prev = acc_ref[:, 0:1]
    m_new = jnp.maximum(m_prev, tile_m)
    acc_ref[:, 1:2] = acc_ref[:, 1:2] * jnp.exp(m_prev - m_new) + jnp.sum(
        jnp.exp(logits - m_new), axis=1, keepdims=True
    )
    acc_ref[:, 0:1] = m_new

    col0 = my_x * V_LOC + q_ref[0] * Q + j * V_TILE
    cols = col0 + lax.broadcasted_iota(jnp.int32, (T, V_TILE), 1)
    hit = cols == lab_ref[...]
    acc_ref[:, 2:3] += jnp.sum(jnp.where(hit, logits, 0.0), axis=1, keepdims=True)


def _allreduce_body(stats_ref, out_ref, send_ref, recv_ref, send_sems, recv_sems):
    my_x = lax.axis_index("x")
    my_y = lax.axis_index("y")
    my_z = lax.axis_index("z")
    partners = [
        (1 - my_x, my_y, my_z),
        (my_x, 1 - my_y, my_z),
        (my_x, my_y, 1 - my_z),
    ]

    rowstats = jnp.transpose(stats_ref[...], (1, 0))

    barrier_sem = pltpu.get_barrier_semaphore()
    for p in partners:
        pl.semaphore_signal(
            barrier_sem, inc=1, device_id=p,
            device_id_type=pl.DeviceIdType.MESH,
        )
    pl.semaphore_wait(barrier_sem, 3)

    m = rowstats[0:1, :]
    l = rowstats[1:2, :]
    g = rowstats[2:3, :]

    for k, p in enumerate(partners):
        send_ref[0:1, :] = m
        send_ref[1:2, :] = l
        send_ref[2:3, :] = g
        rdma = pltpu.make_async_remote_copy(
            src_ref=send_ref,
            dst_ref=recv_ref.at[k],
            send_sem=send_sems.at[k],
            recv_sem=recv_sems.at[k],
            device_id=p,
            device_id_type=pl.DeviceIdType.MESH,
        )
        rdma.start()
        rdma.wait()

        pm = recv_ref[k, 0:1, :]
        pl_ = recv_ref[k, 1:2, :]
        pg = recv_ref[k, 2:3, :]
        m_new = jnp.maximum(m, pm)
        l = l * jnp.exp(m - m_new) + pl_ * jnp.exp(pm - m_new)
        g = g + pg
        m = m_new

    out_ref[...] = (m + jnp.log(l)) - g


def kernel(x, W, labels):
    q = (lax.axis_index("y") * 2 + lax.axis_index("z")).astype(jnp.int32)

    stats = pl.pallas_call(
        _stats_body,
        grid_spec=pltpu.PrefetchScalarGridSpec(
            num_scalar_prefetch=1,
            grid=(N_TILES,),
            in_specs=[
                pl.BlockSpec((T, D), lambda j, q_ref: (0, 0)),
                pl.BlockSpec(
                    (D, V_TILE), lambda j, q_ref: (0, q_ref[0] * N_TILES + j)
                ),
                pl.BlockSpec((T, 1), lambda j, q_ref: (0, 0)),
            ],
            out_specs=pl.BlockSpec((T, 8), lambda j, q_ref: (0, 0)),
            scratch_shapes=[
                pltpu.VMEM((T, D), jnp.bfloat16),
            ],
        ),
        out_shape=jax.ShapeDtypeStruct((T, 8), jnp.float32),
        compiler_params=pltpu.CompilerParams(
            vmem_limit_bytes=128 * 1024 * 1024,
        ),
    )(q.reshape(1), x, W, labels.reshape(T, 1))

    nll = pl.pallas_call(
        _allreduce_body,
        in_specs=[pl.BlockSpec(memory_space=pltpu.VMEM)],
        out_specs=pl.BlockSpec(memory_space=pltpu.VMEM),
        out_shape=jax.ShapeDtypeStruct((1, T), jnp.float32),
        scratch_shapes=[
            pltpu.VMEM((8, T), jnp.float32),
            pltpu.VMEM((3, 8, T), jnp.float32),
            pltpu.SemaphoreType.DMA((3,)),
            pltpu.SemaphoreType.DMA((3,)),
        ],
        compiler_params=pltpu.CompilerParams(collective_id=0),
    )(stats)

    return nll.reshape(T)


# device time: 94766 ns/iter; 2.2023x vs baseline; 2.2023x over previous
import jax
import jax.numpy as jnp
from jax import lax
from jax.experimental import pallas as pl
from jax.experimental.pallas import tpu as pltpu

T = 2048
D = 4096
V_LOC = 8192
Q = V_LOC // 4
V_TILE = 512
N_TILES = Q // V_TILE


def _stats_body(q_ref, x_ref, w_ref, lab_ref, acc_ref):
    j = pl.program_id(0)
    my_x = lax.axis_index("x")

    @pl.when(j == 0)
    def _():
        acc_ref[...] = jnp.zeros((T, 8), jnp.float32)
        acc_ref[:, 0:1] = jnp.full((T, 1), -jnp.inf, jnp.float32)

    logits = jnp.dot(
        x_ref[...], w_ref[...], preferred_element_type=jnp.float32
    )

    tile_m = jnp.max(logits, axis=1, keepdims=True)
    m_prev = acc_ref[:, 0:1]
    m_new = jnp.maximum(m_prev, tile_m)
    acc_ref[:, 1:2] = acc_ref[:, 1:2] * jnp.exp(m_prev - m_new) + jnp.sum(
        jnp.exp(logits - m_new), axis=1, keepdims=True
    )
    acc_ref[:, 0:1] = m_new

    col0 = my_x * V_LOC + q_ref[0] * Q + j * V_TILE
    cols = col0 + lax.broadcasted_iota(jnp.int32, (T, V_TILE), 1)
    hit = cols == lab_ref[...]
    acc_ref[:, 2:3] += jnp.sum(jnp.where(hit, logits, 0.0), axis=1, keepdims=True)


def _allreduce_body(stats_ref, out_ref, send_ref, recv_ref, send_sems, recv_sems):
    my_x = lax.axis_index("x")
    my_y = lax.axis_index("y")
    my_z = lax.axis_index("z")
    partners = [
        (1 - my_x, my_y, my_z),
        (my_x, 1 - my_y, my_z),
        (my_x, my_y, 1 - my_z),
    ]

    rowstats = jnp.transpose(stats_ref[...], (1, 0))

    barrier_sem = pltpu.get_barrier_semaphore()
    for p in partners:
        pl.semaphore_signal(
            barrier_sem, inc=1, device_id=p,
            device_id_type=pl.DeviceIdType.MESH,
        )
    pl.semaphore_wait(barrier_sem, 3)

    m = rowstats[0:1, :]
    l = rowstats[1:2, :]
    g = rowstats[2:3, :]

    for k, p in enumerate(partners):
        send_ref[0:1, :] = m
        send_ref[1:2, :] = l
        send_ref[2:3, :] = g
        rdma = pltpu.make_async_remote_copy(
            src_ref=send_ref,
            dst_ref=recv_ref.at[k],
            send_sem=send_sems.at[k],
            recv_sem=recv_sems.at[k],
            device_id=p,
            device_id_type=pl.DeviceIdType.MESH,
        )
        rdma.start()
        rdma.wait()

        pm = recv_ref[k, 0:1, :]
        pl_ = recv_ref[k, 1:2, :]
        pg = recv_ref[k, 2:3, :]
        m_new = jnp.maximum(m, pm)
        l = l * jnp.exp(m - m_new) + pl_ * jnp.exp(pm - m_new)
        g = g + pg
        m = m_new

    out_ref[...] = (m + jnp.log(l)) - g


def kernel(x, W, labels):
    q = (lax.axis_index("y") * 2 + lax.axis_index("z")).astype(jnp.int32)

    stats = pl.pallas_call(
        _stats_body,
        grid_spec=pltpu.PrefetchScalarGridSpec(
            num_scalar_prefetch=1,
            grid=(N_TILES,),
            in_specs=[
                pl.BlockSpec((T, D), lambda j, q_ref: (0, 0)),
                pl.BlockSpec(
                    (D, V_TILE), lambda j, q_ref: (0, q_ref[0] * N_TILES + j)
                ),
                pl.BlockSpec((T, 1), lambda j, q_ref: (0, 0)),
            ],
            out_specs=pl.BlockSpec((T, 8), lambda j, q_ref: (0, 0)),
        ),
        out_shape=jax.ShapeDtypeStruct((T, 8), jnp.float32),
        compiler_params=pltpu.CompilerParams(
            vmem_limit_bytes=128 * 1024 * 1024,
        ),
    )(q.reshape(1), x, W, labels.reshape(T, 1))

    nll = pl.pallas_call(
        _allreduce_body,
        in_specs=[pl.BlockSpec(memory_space=pltpu.VMEM)],
        out_specs=pl.BlockSpec(memory_space=pltpu.VMEM),
        out_shape=jax.ShapeDtypeStruct((1, T), jnp.float32),
        scratch_shapes=[
            pltpu.VMEM((8, T), jnp.float32),
            pltpu.VMEM((3, 8, T), jnp.float32),
            pltpu.SemaphoreType.DMA((3,)),
            pltpu.SemaphoreType.DMA((3,)),
        ],
        compiler_params=pltpu.CompilerParams(collective_id=0),
    )(stats)

    return nll.reshape(T)
